# Initial kernel scaffold; baseline (speedup 1.0000x reference)
import jax
import jax.numpy as jnp
from jax.experimental import pallas as pl


def kernel(x, edge_index, Wself1, Wneigh1, b1, Wself2, Wneigh2, b2, Wself3, Wneigh3, b3):
    raise NotImplementedError("write your pallas kernel here")


# speedup vs baseline: 5.4394x; 5.4394x over previous
import functools

import jax
import jax.numpy as jnp
from jax import lax
from jax.experimental import pallas as pl
from jax.experimental.pallas import tpu as pltpu
from jax.experimental.pallas import tpu_sc as plsc

N = 10000
NC, NS = 2, 16
NW = NC * NS
CHUNK = 128
NPAD = N + 8
ZROWS = 640
F32 = jnp.float32
BN = 1000


def _sc_segsum(table, srcp, dstp, zeros, zeros8, ones8, *, nchunks, npass,
               with_deg, interpret=False):
    feat = 64
    nplanes = npass * NC
    mesh = plsc.VectorSubcoreMesh(core_axis_name="c", subcore_axis_name="s",
                                  num_cores=NC, num_subcores=NS)
    out_type = [jax.ShapeDtypeStruct((nplanes, N, feat), F32)]
    if with_deg:
        out_type.append(jax.ShapeDtypeStruct((N, 8), F32))
    scratch = [
        pltpu.VMEM((nchunks, CHUNK), jnp.int32),
        pltpu.VMEM((nchunks, CHUNK), jnp.int32),
        pltpu.VMEM((CHUNK, feat), F32),
        pltpu.VMEM((CHUNK, feat), F32),
        pltpu.VMEM((CHUNK, 8), F32),
        pltpu.VMEM_SHARED((NPAD, feat), F32),
        pltpu.VMEM_SHARED((NPAD, 8), F32),
        pltpu.SemaphoreType.DMA,
        pltpu.SemaphoreType.DMA,
    ]

    def body(table_ref, srcp_ref, dstp_ref, zeros_ref, zeros8_ref, ones8_ref,
             *rest):
        if with_deg:
            acc_out, deg_out = rest[0], rest[1]
            rest = rest[2:]
        else:
            acc_out, deg_out = rest[0], None
            rest = rest[1:]
        (src_v, dst_v, rows_a, rows_b, ones_v, acc_sh, deg_sh,
         sem_a, sem_b) = rest
        cid = lax.axis_index("c")
        sid = lax.axis_index("s")
        r0 = sid * ZROWS
        nrows_full = ZROWS
        nrows_last = N - (NS - 1) * ZROWS

        if with_deg:
            pltpu.sync_copy(ones8_ref, ones_v)

        for p in range(npass):
            plane = npass * cid + p

            @pl.when(sid < NS - 1)
            def _():
                pltpu.sync_copy(zeros_ref, acc_sh.at[pl.ds(r0, nrows_full)])
                if with_deg and p == 0:
                    @pl.when(cid == 0)
                    def _():
                        pltpu.sync_copy(zeros8_ref,
                                        deg_sh.at[pl.ds(r0, nrows_full)])

            @pl.when(sid == NS - 1)
            def _():
                b0 = (NS - 1) * ZROWS
                pltpu.sync_copy(zeros_ref.at[pl.ds(0, nrows_last)],
                                acc_sh.at[pl.ds(b0, nrows_last)])
                if with_deg and p == 0:
                    @pl.when(cid == 0)
                    def _():
                        pltpu.sync_copy(zeros8_ref.at[pl.ds(0, nrows_last)],
                                        deg_sh.at[pl.ds(b0, nrows_last)])

            pltpu.sync_copy(srcp_ref.at[plane, sid], src_v)
            pltpu.sync_copy(dstp_ref.at[plane, sid], dst_v)

            plsc.subcore_barrier()

            def gather_start(j, buf, sem):
                pltpu.make_async_copy(
                    table_ref.at[src_v.at[j]], buf, sem).start()

            def gather_wait(j, buf, sem):
                pltpu.make_async_copy(
                    table_ref.at[src_v.at[j]], buf, sem).wait()

            def scatter(j, buf):
                pltpu.sync_copy(buf, acc_sh.at[dst_v.at[j]], add=True)
                if with_deg and p == 0:
                    @pl.when(cid == 0)
                    def _():
                        pltpu.sync_copy(ones_v, deg_sh.at[dst_v.at[j]],
                                        add=True)

            gather_start(0, rows_a, sem_a)

            def loop_body(k, carry):
                j0 = 2 * k
                j1 = j0 + 1
                gather_wait(j0, rows_a, sem_a)
                gather_start(j1, rows_b, sem_b)
                scatter(j0, rows_a)
                gather_wait(j1, rows_b, sem_b)

                @pl.when(j1 + 1 < nchunks)
                def _():
                    gather_start(j1 + 1, rows_a, sem_a)

                scatter(j1, rows_b)
                return carry

            lax.fori_loop(0, nchunks // 2, loop_body, 0)
            plsc.subcore_barrier()

            @pl.when(sid < NS - 1)
            def _():
                pltpu.sync_copy(acc_sh.at[pl.ds(r0, nrows_full)],
                                acc_out.at[plane, pl.ds(r0, nrows_full)])
                if with_deg and p == 0:
                    @pl.when(cid == 0)
                    def _():
                        pltpu.sync_copy(deg_sh.at[pl.ds(r0, nrows_full)],
                                        deg_out.at[pl.ds(r0, nrows_full)])

            @pl.when(sid == NS - 1)
            def _():
                b0 = (NS - 1) * ZROWS
                pltpu.sync_copy(acc_sh.at[pl.ds(b0, nrows_last)],
                                acc_out.at[plane, pl.ds(b0, nrows_last)])
                if with_deg and p == 0:
                    @pl.when(cid == 0)
                    def _():
                        pltpu.sync_copy(deg_sh.at[pl.ds(b0, nrows_last)],
                                        deg_out.at[pl.ds(b0, nrows_last)])

    fn = pl.kernel(
        body, out_type=tuple(out_type), mesh=mesh, scratch_types=scratch,
        compiler_params=pltpu.CompilerParams(use_tc_tiling_on_sc=False),
        interpret=interpret)
    return fn(table, srcp, dstp, zeros, zeros8, ones8)


def _rdeg(deg_blk):
    return 1.0 / jnp.maximum(deg_blk[:, 0:1], 1.0)


def _tc1_body(x_ref, acc_ref, deg_ref, ws_ref, wn_ref, b_ref, out_ref):
    agg = jnp.concatenate([acc_ref[0], acc_ref[1]], axis=1)
    hn = agg * _rdeg(deg_ref[...])
    y = (jnp.dot(x_ref[...], ws_ref[...], preferred_element_type=F32)
         + jnp.dot(hn, wn_ref[...], preferred_element_type=F32)
         + b_ref[...][None, :])
    y = jnp.maximum(y, 0.0)
    for q in range(4):
        out_ref[q] = y[:, q * 64:(q + 1) * 64]


def _tc2_body(h1_ref, acc2_ref, deg_ref, ws2_ref, wn2_ref, b2_ref,
              ws3_ref, wn3_ref, b3_ref, s_ref, p_ref):
    h1 = jnp.concatenate([h1_ref[i] for i in range(4)], axis=1)
    agg = jnp.concatenate([acc2_ref[i] for i in range(4)], axis=1)
    hn = agg * _rdeg(deg_ref[...])
    h2 = (jnp.dot(h1, ws2_ref[...], preferred_element_type=F32)
          + jnp.dot(hn, wn2_ref[...], preferred_element_type=F32)
          + b2_ref[...][None, :])
    h2 = jnp.maximum(h2, 0.0)
    s_ref[...] = (jnp.dot(h2, ws3_ref[...], preferred_element_type=F32)
                  + b3_ref[...][None, :])
    p_ref[...] = jnp.dot(h2, wn3_ref[...], preferred_element_type=F32)


def _tc3_body(s_ref, acc_ref, deg_ref, out_ref):
    agg = acc_ref[0] + acc_ref[1]
    out_ref[...] = s_ref[...] + agg * _rdeg(deg_ref[...])


def _tc1(x, acc1, deg, ws, wn, b):
    return pl.pallas_call(
        _tc1_body,
        grid=(N // BN,),
        in_specs=[
            pl.BlockSpec((BN, 128), lambda i: (i, 0)),
            pl.BlockSpec((2, BN, 64), lambda i: (0, i, 0)),
            pl.BlockSpec((BN, 8), lambda i: (i, 0)),
            pl.BlockSpec((128, 256), lambda i: (0, 0)),
            pl.BlockSpec((128, 256), lambda i: (0, 0)),
            pl.BlockSpec((256,), lambda i: (0,)),
        ],
        out_specs=pl.BlockSpec((4, BN, 64), lambda i: (0, i, 0)),
        out_shape=jax.ShapeDtypeStruct((4, N, 64), F32),
    )(x, acc1, deg, ws, wn, b)


def _tc2(h1q, acc2, deg, ws2, wn2, b2, ws3, wn3, b3):
    return pl.pallas_call(
        _tc2_body,
        grid=(N // BN,),
        in_specs=[
            pl.BlockSpec((4, BN, 64), lambda i: (0, i, 0)),
            pl.BlockSpec((4, BN, 64), lambda i: (0, i, 0)),
            pl.BlockSpec((BN, 8), lambda i: (i, 0)),
            pl.BlockSpec((256, 256), lambda i: (0, 0)),
            pl.BlockSpec((256, 256), lambda i: (0, 0)),
            pl.BlockSpec((256,), lambda i: (0,)),
            pl.BlockSpec((256, 64), lambda i: (0, 0)),
            pl.BlockSpec((256, 64), lambda i: (0, 0)),
            pl.BlockSpec((64,), lambda i: (0,)),
        ],
        out_specs=[
            pl.BlockSpec((BN, 64), lambda i: (i, 0)),
            pl.BlockSpec((BN, 64), lambda i: (i, 0)),
        ],
        out_shape=[
            jax.ShapeDtypeStruct((N, 64), F32),
            jax.ShapeDtypeStruct((N, 64), F32),
        ],
    )(h1q, acc2, deg, ws2, wn2, b2, ws3, wn3, b3)


def _tc3(s, acc3, deg):
    return pl.pallas_call(
        _tc3_body,
        grid=(N // BN,),
        in_specs=[
            pl.BlockSpec((BN, 64), lambda i: (i, 0)),
            pl.BlockSpec((2, BN, 64), lambda i: (0, i, 0)),
            pl.BlockSpec((BN, 8), lambda i: (i, 0)),
        ],
        out_specs=pl.BlockSpec((BN, 64), lambda i: (i, 0)),
        out_shape=jax.ShapeDtypeStruct((N, 64), F32),
    )(s, acc3, deg)


def _even_chunks(edges_per_tile):
    nc = -(-edges_per_tile // CHUNK)
    return nc + (nc % 2)


def kernel(x, edge_index, Wself1, Wneigh1, b1, Wself2, Wneigh2, b2,
           Wself3, Wneigh3, b3):
    E = edge_index.shape[1]
    src = edge_index[0].astype(jnp.int32)
    dst = edge_index[1].astype(jnp.int32)

    eps = E // NS
    npc_f = _even_chunks(eps)
    padf = npc_f * CHUNK - eps
    src_f = jnp.concatenate(
        [src.reshape(NS, eps), jnp.zeros((NS, padf), jnp.int32)], axis=1)
    dst_f = jnp.concatenate(
        [dst.reshape(NS, eps), jnp.full((NS, padf), N, jnp.int32)], axis=1)
    offs2 = (jnp.arange(2, dtype=jnp.int32) * N)[:, None, None]
    src_h = (src_f[None] + offs2).reshape(2, NS, npc_f, CHUNK)
    dst_h = jnp.broadcast_to(dst_f, (2, NS, npc_f * CHUNK)).reshape(
        2, NS, npc_f, CHUNK)
    offs4 = (jnp.arange(4, dtype=jnp.int32) * N)[:, None, None]
    src_q = (src_f[None] + offs4).reshape(4, NS, npc_f, CHUNK)
    dst_q = jnp.broadcast_to(dst_f, (4, NS, npc_f * CHUNK)).reshape(
        4, NS, npc_f, CHUNK)

    epw = E // NW
    npc_e = _even_chunks(epw)
    pade = npc_e * CHUNK - epw
    src_e = jnp.concatenate(
        [src.reshape(NW, epw), jnp.zeros((NW, pade), jnp.int32)], axis=1
    ).reshape(2, NS, npc_e, CHUNK)
    dst_e = jnp.concatenate(
        [dst.reshape(NW, epw), jnp.full((NW, pade), N, jnp.int32)], axis=1
    ).reshape(2, NS, npc_e, CHUNK)

    z64 = jnp.zeros((ZROWS, 64), F32)
    z8 = jnp.zeros((ZROWS, 8), F32)
    ones8 = jnp.ones((CHUNK, 8), F32)

    xs = jnp.concatenate([x[:, :64], x[:, 64:]], axis=0)
    acc1, deg = _sc_segsum(xs, src_h, dst_h, z64, z8, ones8,
                           nchunks=npc_f, npass=1, with_deg=True)
    h1q = _tc1(x, acc1, deg, Wself1, Wneigh1, b1)

    table2 = h1q.reshape(4 * N, 64)
    (acc2,) = _sc_segsum(table2, src_q, dst_q, z64, z8, ones8,
                         nchunks=npc_f, npass=2, with_deg=False)
    s, p = _tc2(h1q, acc2, deg, Wself2, Wneigh2, b2, Wself3, Wneigh3, b3)

    (acc3,) = _sc_segsum(p, src_e, dst_e, z64, z8, ones8,
                         nchunks=npc_e, npass=1, with_deg=False)
    return _tc3(s, acc3, deg)
